# Initial kernel scaffold; baseline (speedup 1.0000x reference)
import jax
import jax.numpy as jnp
from jax.experimental import pallas as pl


def kernel(x, W_qkv, W_proj, b_proj, islast):
    raise NotImplementedError("write your pallas kernel here")


# speedup vs baseline: 19.8801x; 19.8801x over previous
import jax
import jax.numpy as jnp
from jax.experimental import pallas as pl
from jax.experimental.pallas import tpu as pltpu

_DIM = 768
_H = 12
_K = 90
_B = 8
_N = 576
_HD = _DIM // _H
_SCALE = _HD ** -0.5
_MININT = -(2 ** 31)


def _qkv_kernel(x_ref, w_ref, o_ref):
    o_ref[0, 0] = jax.lax.dot_general(
        x_ref[0], w_ref[...],
        dimension_numbers=(((1,), (1,)), ((), ())),
        preferred_element_type=jnp.float32)


def _attn_kernel(islast_ref, q_ref, k_ref, v_ref, attn_ref, ho_ref):
    q = q_ref[0, 0]
    k = k_ref[0, 0]
    v = v_ref[0, 0]
    s = jax.lax.dot_general(
        q, k, dimension_numbers=(((1,), (1,)), ((), ())),
        preferred_element_type=jnp.float32) * _SCALE

    bits = jax.lax.bitcast_convert_type(s, jnp.int32)
    u = jnp.where(bits >= 0, bits ^ _MININT, ~bits)

    def body(i, carry):
        prefix, krem = carry
        b = 31 - i
        bitv = jnp.int32(1) << b
        hm = jnp.int32(-1) << b
        cand = prefix | bitv
        eq = (u & hm) == cand
        c = jnp.sum(eq.astype(jnp.int32), axis=1, keepdims=True)
        ge = c >= krem
        prefix = jnp.where(ge, cand, prefix)
        krem = jnp.where(ge, krem, krem - c)
        return prefix, krem

    prefix0 = jnp.zeros((_N, 1), jnp.int32)
    krem0 = jnp.full((_N, 1), _K, jnp.int32)
    prefix, _ = jax.lax.fori_loop(0, 32, body, (prefix0, krem0))

    su = u ^ _MININT
    sthr = prefix ^ _MININT
    sthr = jnp.where(islast_ref[0] == 0, sthr, _MININT)
    mask = su >= sthr

    m = jnp.max(s, axis=1, keepdims=True)
    p = jnp.where(mask, jnp.exp(s - m), 0.0)
    a = p / jnp.sum(p, axis=1, keepdims=True)
    attn_ref[0, 0] = a
    ho_ref[0, 0] = jax.lax.dot_general(
        a, v, dimension_numbers=(((1,), (0,)), ((), ())),
        preferred_element_type=jnp.float32)


def _proj_kernel(ho_ref, w_ref, b_ref, o_ref):
    o_ref[0] = jax.lax.dot_general(
        ho_ref[0], w_ref[...],
        dimension_numbers=(((1,), (1,)), ((), ())),
        preferred_element_type=jnp.float32) + b_ref[...]


def kernel(x, W_qkv, W_proj, b_proj, islast):
    islast_arr = jnp.asarray(islast, jnp.int32).reshape(1)

    qkv = pl.pallas_call(
        _qkv_kernel,
        grid=(_B, 3 * _H),
        in_specs=[
            pl.BlockSpec((1, _N, _DIM), lambda b, g: (b, 0, 0)),
            pl.BlockSpec((_HD, _DIM), lambda b, g: (g, 0)),
        ],
        out_specs=pl.BlockSpec((1, 1, _N, _HD), lambda b, g: (b, g, 0, 0)),
        out_shape=jax.ShapeDtypeStruct((_B, 3 * _H, _N, _HD), jnp.float32),
        compiler_params=pltpu.CompilerParams(
            dimension_semantics=("parallel", "parallel")),
    )(x, W_qkv)

    attn, ho = pl.pallas_call(
        _attn_kernel,
        grid=(_B, _H),
        in_specs=[
            pl.BlockSpec(memory_space=pltpu.SMEM),
            pl.BlockSpec((1, 1, _N, _HD), lambda b, h: (b, h, 0, 0)),
            pl.BlockSpec((1, 1, _N, _HD), lambda b, h: (b, _H + h, 0, 0)),
            pl.BlockSpec((1, 1, _N, _HD), lambda b, h: (b, 2 * _H + h, 0, 0)),
        ],
        out_specs=[
            pl.BlockSpec((1, 1, _N, _N), lambda b, h: (b, h, 0, 0)),
            pl.BlockSpec((1, 1, _N, _HD), lambda b, h: (b, h, 0, 0)),
        ],
        out_shape=[
            jax.ShapeDtypeStruct((_B, _H, _N, _N), jnp.float32),
            jax.ShapeDtypeStruct((_B, _H, _N, _HD), jnp.float32),
        ],
        compiler_params=pltpu.CompilerParams(
            dimension_semantics=("parallel", "parallel")),
    )(islast_arr, qkv, qkv, qkv)

    ho_bnc = ho.transpose(0, 2, 1, 3).reshape(_B, _N, _DIM)

    out = pl.pallas_call(
        _proj_kernel,
        grid=(_B,),
        in_specs=[
            pl.BlockSpec((1, _N, _DIM), lambda b: (b, 0, 0)),
            pl.BlockSpec((_DIM, _DIM), lambda b: (0, 0)),
            pl.BlockSpec((1, _DIM), lambda b: (0, 0)),
        ],
        out_specs=pl.BlockSpec((1, _N, _DIM), lambda b: (b, 0, 0)),
        out_shape=jax.ShapeDtypeStruct((_B, _N, _DIM), jnp.float32),
        compiler_params=pltpu.CompilerParams(
            dimension_semantics=("parallel",)),
    )(ho_bnc, W_proj, b_proj.reshape(1, _DIM))

    return (out, attn)
